# baseline (speedup 1.0000x reference)
import functools

import jax
import jax.numpy as jnp
from jax import lax
from jax.experimental import pallas as pl
from jax.experimental.pallas import tpu as pltpu
from jax.experimental.pallas import tpu_sc as plsc

N_CLASSES = 100000
EMBED_DIM = 64
BATCH = 16384

NW = 32
CHUNK = 128
NCHI = BATCH // CHUNK
NBUF = 4

S = 8
JC = NW // S
QW = NW // JC
CQ = NCHI // QW

TRB = 4096


def _sigma_body(wt_ref, ut_ref, o_ref, g_acc, v_acc):
    wt = wt_ref[...]
    g_acc[...] = jax.lax.dot_general(
        wt, wt, (((1,), (1,)), ((), ())),
        preferred_element_type=jnp.float32,
        precision=jax.lax.Precision.HIGHEST,
    )
    v_acc[...] = jnp.sum(wt * ut_ref[...], axis=1, keepdims=True)
    v = v_acc[...]
    v_hat = v * lax.rsqrt(jnp.maximum(jnp.sum(v * v), 1e-12))
    gv = jax.lax.dot_general(
        g_acc[...], v_hat, (((1,), (0,)), ((), ())),
        preferred_element_type=jnp.float32,
        precision=jax.lax.Precision.HIGHEST,
    )
    s2 = jnp.sum(gv * v_hat)
    sigma = s2 * lax.rsqrt(jnp.maximum(s2, 1e-12))
    o_ref[0, 0] = 1.0 / sigma


def _sigma_call(w_t, u_t):
    return pl.pallas_call(
        _sigma_body,
        grid=(1,),
        in_specs=[
            pl.BlockSpec((EMBED_DIM, N_CLASSES), lambda i: (0, 0)),
            pl.BlockSpec((1, N_CLASSES), lambda i: (0, 0)),
        ],
        out_specs=pl.BlockSpec(memory_space=pltpu.SMEM),
        out_shape=jax.ShapeDtypeStruct((1, 1), jnp.float32),
        scratch_shapes=[
            pltpu.VMEM((EMBED_DIM, EMBED_DIM), jnp.float32),
            pltpu.VMEM((EMBED_DIM, 1), jnp.float32),
        ],
        compiler_params=pltpu.CompilerParams(
            vmem_limit_bytes=100 * 1024 * 1024),
    )(w_t, u_t)


def _gather_body(w_hbm, labels_hbm, out_hbm, idx_v, rows_v, sems):
    wid = lax.axis_index("s") * 2 + lax.axis_index("c")
    jl = wid // QW
    q = lax.rem(wid, QW)

    pltpu.sync_copy(
        labels_hbm.at[pl.ds(2 * jl, 2), pl.ds(q * CQ, CQ)], idx_v)

    for p in range(2):
        for b in range(NBUF):
            pltpu.async_copy(
                w_hbm.at[idx_v.at[p, b]], rows_v.at[b], sems.at[b])

        @pl.loop(0, CQ - NBUF, step=NBUF)
        def _(g0):
            for b in range(NBUF):
                g = g0 + b
                pltpu.make_async_copy(
                    w_hbm.at[idx_v.at[p, g]], rows_v.at[b], sems.at[b]).wait()
                pltpu.sync_copy(
                    rows_v.at[b],
                    out_hbm.at[jl, pl.ds((q * CQ + g) * CHUNK, CHUNK),
                               pl.ds(p * EMBED_DIM, EMBED_DIM)])
                pltpu.async_copy(
                    w_hbm.at[idx_v.at[p, g + NBUF]], rows_v.at[b], sems.at[b])

        for b in range(NBUF):
            g = CQ - NBUF + b
            pltpu.make_async_copy(
                w_hbm.at[idx_v.at[p, g]], rows_v.at[b], sems.at[b]).wait()
            pltpu.sync_copy(
                rows_v.at[b],
                out_hbm.at[jl, pl.ds((q * CQ + g) * CHUNK, CHUNK),
                           pl.ds(p * EMBED_DIM, EMBED_DIM)])


_gather_call = functools.partial(
    pl.kernel,
    out_type=jax.ShapeDtypeStruct((JC, BATCH, 2 * EMBED_DIM), jnp.float32),
    mesh=plsc.VectorSubcoreMesh(core_axis_name="c", subcore_axis_name="s"),
    scratch_types=[
        pltpu.VMEM((2, CQ, CHUNK), jnp.int32),
        pltpu.VMEM((NBUF, CHUNK, EMBED_DIM), jnp.float32),
        pltpu.SemaphoreType.DMA((NBUF,)),
    ],
    compiler_params=pltpu.CompilerParams(use_tc_tiling_on_sc=False),
)(_gather_body)


def _transpose_body(recip_ref, y_ref, *rest):
    o_ref = rest[-1]
    a = y_ref[0]
    r = recip_ref[0, 0]
    o_ref[0] = jnp.swapaxes(a[:, 0:EMBED_DIM], 0, 1) * r
    o_ref[1] = jnp.swapaxes(a[:, EMBED_DIM:2 * EMBED_DIM], 0, 1) * r


def _transpose_chunk(y2_s, recip, s, xt_prev):
    out_shape = jax.ShapeDtypeStruct((EMBED_DIM, EMBED_DIM, BATCH),
                                     jnp.float32)
    out_spec = pl.BlockSpec(
        (2, EMBED_DIM, TRB), lambda j, t, _s=s: (_s * JC + j, 0, t))
    in_specs = [
        pl.BlockSpec(memory_space=pltpu.SMEM),
        pl.BlockSpec((1, TRB, 2 * EMBED_DIM), lambda j, t: (j, t, 0)),
    ]
    args = [recip, y2_s]
    kwargs = {}
    if xt_prev is not None:
        in_specs.append(pl.BlockSpec(memory_space=pl.ANY))
        args.append(xt_prev)
        kwargs["input_output_aliases"] = {2: 0}
    return pl.pallas_call(
        _transpose_body,
        grid=(JC, BATCH // TRB),
        in_specs=in_specs,
        out_specs=out_spec,
        out_shape=out_shape,
        **kwargs,
    )(*args)


def kernel(labels, w, u):
    w_t = w.T
    u_t = u.T
    recip = _sigma_call(w_t, u_t)
    labels3 = labels.T.reshape(EMBED_DIM, NCHI, CHUNK)

    xt = None
    for s in range(S):
        labels_s = labels3[2 * s * JC:2 * (s + 1) * JC]
        y2_s = _gather_call(w, labels_s)
        xt = _transpose_chunk(y2_s, recip, s, xt)
    return xt.transpose(2, 0, 1)


# speedup vs baseline: 1.0827x; 1.0827x over previous
import functools

import jax
import jax.numpy as jnp
from jax import lax
from jax.experimental import pallas as pl
from jax.experimental.pallas import tpu as pltpu
from jax.experimental.pallas import tpu_sc as plsc

N_CLASSES = 100000
EMBED_DIM = 64
BATCH = 16384

NW = 32
CHUNK = 128
NCHI = BATCH // CHUNK
NBUF = 4

S = 4
JC = NW // S
QW = NW // JC
CQ = NCHI // QW

TRB = 8192


def _sigma_body(wt_ref, ut_ref, o_ref, g_acc, v_acc):
    wt = wt_ref[...]
    g_acc[...] = jax.lax.dot_general(
        wt, wt, (((1,), (1,)), ((), ())),
        preferred_element_type=jnp.float32,
        precision=jax.lax.Precision.HIGHEST,
    )
    v_acc[...] = jnp.sum(wt * ut_ref[...], axis=1, keepdims=True)
    v = v_acc[...]
    v_hat = v * lax.rsqrt(jnp.maximum(jnp.sum(v * v), 1e-12))
    gv = jax.lax.dot_general(
        g_acc[...], v_hat, (((1,), (0,)), ((), ())),
        preferred_element_type=jnp.float32,
        precision=jax.lax.Precision.HIGHEST,
    )
    s2 = jnp.sum(gv * v_hat)
    sigma = s2 * lax.rsqrt(jnp.maximum(s2, 1e-12))
    o_ref[0, 0] = 1.0 / sigma


def _sigma_call(w_t, u_t):
    return pl.pallas_call(
        _sigma_body,
        grid=(1,),
        in_specs=[
            pl.BlockSpec((EMBED_DIM, N_CLASSES), lambda i: (0, 0)),
            pl.BlockSpec((1, N_CLASSES), lambda i: (0, 0)),
        ],
        out_specs=pl.BlockSpec(memory_space=pltpu.SMEM),
        out_shape=jax.ShapeDtypeStruct((1, 1), jnp.float32),
        scratch_shapes=[
            pltpu.VMEM((EMBED_DIM, EMBED_DIM), jnp.float32),
            pltpu.VMEM((EMBED_DIM, 1), jnp.float32),
        ],
        compiler_params=pltpu.CompilerParams(
            vmem_limit_bytes=100 * 1024 * 1024),
    )(w_t, u_t)


def _gather_body(w_hbm, labels_hbm, out_hbm, idx_v, rows_v, sems):
    wid = lax.axis_index("s") * 2 + lax.axis_index("c")
    jl = wid // QW
    q = lax.rem(wid, QW)

    pltpu.sync_copy(
        labels_hbm.at[pl.ds(2 * jl, 2), pl.ds(q * CQ, CQ)], idx_v)

    for p in range(2):
        for b in range(NBUF):
            pltpu.async_copy(
                w_hbm.at[idx_v.at[p, b]], rows_v.at[b], sems.at[b])

        @pl.loop(0, CQ - NBUF, step=NBUF)
        def _(g0):
            for b in range(NBUF):
                g = g0 + b
                pltpu.make_async_copy(
                    w_hbm.at[idx_v.at[p, g]], rows_v.at[b], sems.at[b]).wait()
                pltpu.sync_copy(
                    rows_v.at[b],
                    out_hbm.at[jl, pl.ds((q * CQ + g) * CHUNK, CHUNK),
                               pl.ds(p * EMBED_DIM, EMBED_DIM)])
                pltpu.async_copy(
                    w_hbm.at[idx_v.at[p, g + NBUF]], rows_v.at[b], sems.at[b])

        for b in range(NBUF):
            g = CQ - NBUF + b
            pltpu.make_async_copy(
                w_hbm.at[idx_v.at[p, g]], rows_v.at[b], sems.at[b]).wait()
            pltpu.sync_copy(
                rows_v.at[b],
                out_hbm.at[jl, pl.ds((q * CQ + g) * CHUNK, CHUNK),
                           pl.ds(p * EMBED_DIM, EMBED_DIM)])


_gather_call = functools.partial(
    pl.kernel,
    out_type=jax.ShapeDtypeStruct((JC, BATCH, 2 * EMBED_DIM), jnp.float32),
    mesh=plsc.VectorSubcoreMesh(core_axis_name="c", subcore_axis_name="s"),
    scratch_types=[
        pltpu.VMEM((2, CQ, CHUNK), jnp.int32),
        pltpu.VMEM((NBUF, CHUNK, EMBED_DIM), jnp.float32),
        pltpu.SemaphoreType.DMA((NBUF,)),
    ],
    compiler_params=pltpu.CompilerParams(use_tc_tiling_on_sc=False),
)(_gather_body)


def _transpose_body(recip_ref, y_ref, *rest):
    o_ref = rest[-1]
    a = y_ref[0]
    r = recip_ref[0, 0]
    o_ref[0] = jnp.swapaxes(a[:, 0:EMBED_DIM], 0, 1) * r
    o_ref[1] = jnp.swapaxes(a[:, EMBED_DIM:2 * EMBED_DIM], 0, 1) * r


def _transpose_chunk(y2_s, recip, s, xt_prev):
    out_shape = jax.ShapeDtypeStruct((EMBED_DIM, EMBED_DIM, BATCH),
                                     jnp.float32)
    out_spec = pl.BlockSpec(
        (2, EMBED_DIM, TRB), lambda j, t, _s=s: (_s * JC + j, 0, t))
    in_specs = [
        pl.BlockSpec(memory_space=pltpu.SMEM),
        pl.BlockSpec((1, TRB, 2 * EMBED_DIM), lambda j, t: (j, t, 0)),
    ]
    args = [recip, y2_s]
    kwargs = {}
    if xt_prev is not None:
        in_specs.append(pl.BlockSpec(memory_space=pl.ANY))
        args.append(xt_prev)
        kwargs["input_output_aliases"] = {2: 0}
    return pl.pallas_call(
        _transpose_body,
        grid=(JC, BATCH // TRB),
        in_specs=in_specs,
        out_specs=out_spec,
        out_shape=out_shape,
        **kwargs,
    )(*args)


def kernel(labels, w, u):
    w_t = w.T
    u_t = u.T
    recip = _sigma_call(w_t, u_t)
    labels3 = labels.T.reshape(EMBED_DIM, NCHI, CHUNK)

    xt = None
    for s in range(S):
        labels_s = labels3[2 * s * JC:2 * (s + 1) * JC]
        y2_s = _gather_call(w, labels_s)
        xt = _transpose_chunk(y2_s, recip, s, xt)
    return xt.transpose(2, 0, 1)


# speedup vs baseline: 1.0979x; 1.0140x over previous
import functools

import jax
import jax.numpy as jnp
from jax import lax
from jax.experimental import pallas as pl
from jax.experimental.pallas import tpu as pltpu
from jax.experimental.pallas import tpu_sc as plsc

N_CLASSES = 100000
EMBED_DIM = 64
BATCH = 16384

NW = 32
CHUNK = 128
NCHI = BATCH // CHUNK
NBUF = 4

S = 4
JC = NW // S
QW = NW // JC
CQ = NCHI // QW

TRB = 16384


def _sigma_body(wt_ref, ut_ref, o_ref, g_acc, v_acc):
    wt = wt_ref[...]
    g_acc[...] = jax.lax.dot_general(
        wt, wt, (((1,), (1,)), ((), ())),
        preferred_element_type=jnp.float32,
        precision=jax.lax.Precision.HIGHEST,
    )
    v_acc[...] = jnp.sum(wt * ut_ref[...], axis=1, keepdims=True)
    v = v_acc[...]
    v_hat = v * lax.rsqrt(jnp.maximum(jnp.sum(v * v), 1e-12))
    gv = jax.lax.dot_general(
        g_acc[...], v_hat, (((1,), (0,)), ((), ())),
        preferred_element_type=jnp.float32,
        precision=jax.lax.Precision.HIGHEST,
    )
    s2 = jnp.sum(gv * v_hat)
    sigma = s2 * lax.rsqrt(jnp.maximum(s2, 1e-12))
    o_ref[0, 0] = 1.0 / sigma


def _sigma_call(w_t, u_t):
    return pl.pallas_call(
        _sigma_body,
        grid=(1,),
        in_specs=[
            pl.BlockSpec((EMBED_DIM, N_CLASSES), lambda i: (0, 0)),
            pl.BlockSpec((1, N_CLASSES), lambda i: (0, 0)),
        ],
        out_specs=pl.BlockSpec(memory_space=pltpu.SMEM),
        out_shape=jax.ShapeDtypeStruct((1, 1), jnp.float32),
        scratch_shapes=[
            pltpu.VMEM((EMBED_DIM, EMBED_DIM), jnp.float32),
            pltpu.VMEM((EMBED_DIM, 1), jnp.float32),
        ],
        compiler_params=pltpu.CompilerParams(
            vmem_limit_bytes=100 * 1024 * 1024),
    )(w_t, u_t)


def _gather_body(w_hbm, labels_hbm, out_hbm, idx_v, rows_v, sems):
    wid = lax.axis_index("s") * 2 + lax.axis_index("c")
    jl = wid // QW
    q = lax.rem(wid, QW)

    pltpu.sync_copy(
        labels_hbm.at[pl.ds(2 * jl, 2), pl.ds(q * CQ, CQ)], idx_v)

    for p in range(2):
        for b in range(NBUF):
            pltpu.async_copy(
                w_hbm.at[idx_v.at[p, b]], rows_v.at[b], sems.at[b])

        @pl.loop(0, CQ - NBUF, step=NBUF)
        def _(g0):
            for b in range(NBUF):
                g = g0 + b
                pltpu.make_async_copy(
                    w_hbm.at[idx_v.at[p, g]], rows_v.at[b], sems.at[b]).wait()
                pltpu.sync_copy(
                    rows_v.at[b],
                    out_hbm.at[jl, pl.ds((q * CQ + g) * CHUNK, CHUNK),
                               pl.ds(p * EMBED_DIM, EMBED_DIM)])
                pltpu.async_copy(
                    w_hbm.at[idx_v.at[p, g + NBUF]], rows_v.at[b], sems.at[b])

        for b in range(NBUF):
            g = CQ - NBUF + b
            pltpu.make_async_copy(
                w_hbm.at[idx_v.at[p, g]], rows_v.at[b], sems.at[b]).wait()
            pltpu.sync_copy(
                rows_v.at[b],
                out_hbm.at[jl, pl.ds((q * CQ + g) * CHUNK, CHUNK),
                           pl.ds(p * EMBED_DIM, EMBED_DIM)])


_gather_call = functools.partial(
    pl.kernel,
    out_type=jax.ShapeDtypeStruct((JC, BATCH, 2 * EMBED_DIM), jnp.float32),
    mesh=plsc.VectorSubcoreMesh(core_axis_name="c", subcore_axis_name="s"),
    scratch_types=[
        pltpu.VMEM((2, CQ, CHUNK), jnp.int32),
        pltpu.VMEM((NBUF, CHUNK, EMBED_DIM), jnp.float32),
        pltpu.SemaphoreType.DMA((NBUF,)),
    ],
    compiler_params=pltpu.CompilerParams(use_tc_tiling_on_sc=False),
)(_gather_body)


def _transpose_body(recip_ref, y_ref, *rest):
    o_ref = rest[-1]
    a = y_ref[0]
    r = recip_ref[0, 0]
    o_ref[0] = jnp.swapaxes(a[:, 0:EMBED_DIM], 0, 1) * r
    o_ref[1] = jnp.swapaxes(a[:, EMBED_DIM:2 * EMBED_DIM], 0, 1) * r


def _transpose_chunk(y2_s, recip, s, xt_prev):
    out_shape = jax.ShapeDtypeStruct((EMBED_DIM, EMBED_DIM, BATCH),
                                     jnp.float32)
    out_spec = pl.BlockSpec(
        (2, EMBED_DIM, TRB), lambda j, t, _s=s: (_s * JC + j, 0, t))
    in_specs = [
        pl.BlockSpec(memory_space=pltpu.SMEM),
        pl.BlockSpec((1, TRB, 2 * EMBED_DIM), lambda j, t: (j, t, 0)),
    ]
    args = [recip, y2_s]
    kwargs = {}
    if xt_prev is not None:
        in_specs.append(pl.BlockSpec(memory_space=pl.ANY))
        args.append(xt_prev)
        kwargs["input_output_aliases"] = {2: 0}
    return pl.pallas_call(
        _transpose_body,
        grid=(JC, BATCH // TRB),
        in_specs=in_specs,
        out_specs=out_spec,
        out_shape=out_shape,
        compiler_params=pltpu.CompilerParams(
            vmem_limit_bytes=100 * 1024 * 1024),
        **kwargs,
    )(*args)


def kernel(labels, w, u):
    w_t = w.T
    u_t = u.T
    recip = _sigma_call(w_t, u_t)
    labels3 = labels.T.reshape(EMBED_DIM, NCHI, CHUNK)

    xt = None
    for s in range(S):
        labels_s = labels3[2 * s * JC:2 * (s + 1) * JC]
        y2_s = _gather_call(w, labels_s)
        xt = _transpose_chunk(y2_s, recip, s, xt)
    return xt.transpose(2, 0, 1)
